# baseline (speedup 1.0000x reference)
import functools
from math import sqrt

import jax
import jax.numpy as jnp
from jax import lax
from jax.experimental import pallas as pl
from jax.experimental.pallas import tpu as pltpu
from jax.experimental.pallas import tpu_sc as plsc

D_MODEL = 64
SCALE = float(sqrt(D_MODEL))

NC = 2
NS = 16
NW = NC * NS
LANES = 16

CHUNK = 256
VECS_PER_ROW = D_MODEL // LANES


def _make_lookup(B, V):
    assert B % NW == 0
    b_per_w = B // NW
    assert b_per_w % CHUNK == 0
    nchunks = b_per_w // CHUNK

    mesh = plsc.VectorSubcoreMesh(
        core_axis_name="c", subcore_axis_name="s",
        num_cores=NC, num_subcores=NS)

    @functools.partial(
        pl.kernel,
        mesh=mesh,
        compiler_params=pltpu.CompilerParams(use_tc_tiling_on_sc=False),
        out_type=jax.ShapeDtypeStruct((B, 2 * D_MODEL), jnp.float32),
        scratch_types=[
            pltpu.VMEM((b_per_w,), jnp.int32),
            pltpu.VMEM((CHUNK, D_MODEL), jnp.float32),
            pltpu.VMEM((CHUNK, D_MODEL), jnp.float32),
            pltpu.VMEM((CHUNK, 2 * D_MODEL), jnp.float32),
            pltpu.SemaphoreType.DMA,
            pltpu.SemaphoreType.DMA,
        ],
    )
    def lookup(x_hbm, table_hbm, out_hbm, idx_v, buf0, buf1, obuf,
               sem0, sem1):
        wid = lax.axis_index("s") * NC + lax.axis_index("c")
        base = wid * b_per_w

        pltpu.sync_copy(x_hbm.at[wid], idx_v)

        zeros = jnp.zeros((LANES,), jnp.float32)

        @pl.loop(0, CHUNK)
        def _zero(i):
            for j in range(VECS_PER_ROW):
                obuf[i, pl.ds(D_MODEL + j * LANES, LANES)] = zeros

        bufs = (buf0, buf1)
        sems = (sem0, sem1)

        def idx_slice(g):
            return idx_v.at[pl.ds(g * CHUNK, CHUNK)]

        def start_gather(g, b):
            pltpu.async_copy(table_hbm.at[idx_slice(g)], bufs[b], sems[b])

        def scale_and_store(g, b):
            buf = bufs[b]
            pltpu.make_async_copy(table_hbm.at[idx_slice(g)], buf,
                                  sems[b]).wait()

            @pl.loop(0, CHUNK)
            def _scale(i):
                for j in range(VECS_PER_ROW):
                    sl = pl.ds(j * LANES, LANES)
                    obuf[i, sl] = buf[i, sl] * SCALE

            pltpu.sync_copy(obuf, out_hbm.at[pl.ds(base + g * CHUNK, CHUNK)])

        start_gather(0, 0)
        start_gather(1, 1)

        @pl.loop(0, nchunks - 2, step=2)
        def _chunks(g0):
            for b in range(2):
                g = g0 + b
                scale_and_store(g, b)
                start_gather(g + 2, b)

        for b in range(2):
            scale_and_store(nchunks - 2 + b, b)

    return lookup


def kernel(X, table):
    rows, cols = X.shape
    B = rows * cols
    V = table.shape[0]
    xf = X.reshape(NW, B // NW).astype(jnp.int32)
    out = _make_lookup(B, V)(xf, table)
    return out[:, :D_MODEL].reshape(rows, cols, D_MODEL)


# speedup vs baseline: 1.4361x; 1.4361x over previous
import functools
from math import sqrt

import jax
import jax.numpy as jnp
from jax import lax
from jax.experimental import pallas as pl
from jax.experimental.pallas import tpu as pltpu
from jax.experimental.pallas import tpu_sc as plsc

D_MODEL = 64
SCALE = float(sqrt(D_MODEL))

NC = 2
NS = 16
NW = NC * NS
LANES = 16

CHUNK = 512
VECS_PER_ROW = D_MODEL // LANES


def _make_lookup(B, V):
    assert B % NW == 0
    b_per_w = B // NW
    assert b_per_w % CHUNK == 0
    nchunks = b_per_w // CHUNK

    mesh = plsc.VectorSubcoreMesh(
        core_axis_name="c", subcore_axis_name="s",
        num_cores=NC, num_subcores=NS)

    @functools.partial(
        pl.kernel,
        mesh=mesh,
        compiler_params=pltpu.CompilerParams(use_tc_tiling_on_sc=False),
        out_type=jax.ShapeDtypeStruct((B, 2 * D_MODEL), jnp.float32),
        scratch_types=[
            pltpu.VMEM((b_per_w,), jnp.int32),
            pltpu.VMEM((CHUNK, D_MODEL), jnp.float32),
            pltpu.VMEM((CHUNK, D_MODEL), jnp.float32),
            pltpu.SemaphoreType.DMA,
            pltpu.SemaphoreType.DMA,
        ],
    )
    def lookup(x_hbm, table_hbm, out_hbm, idx_v, buf0, buf1, sem0, sem1):
        wid = lax.axis_index("s") * NC + lax.axis_index("c")
        base = wid * b_per_w

        pltpu.sync_copy(x_hbm.at[wid], idx_v)

        bufs = (buf0, buf1)
        sems = (sem0, sem1)

        def idx_slice(g):
            return idx_v.at[pl.ds(g * CHUNK, CHUNK)]

        def start_gather(g, b):
            pltpu.async_copy(table_hbm.at[idx_slice(g)], bufs[b], sems[b])

        def scale_and_store(g, b):
            buf = bufs[b]
            pltpu.make_async_copy(table_hbm.at[idx_slice(g)], buf,
                                  sems[b]).wait()

            @pl.loop(0, CHUNK)
            def _scale(i):
                for j in range(VECS_PER_ROW):
                    sl = pl.ds(j * LANES, LANES)
                    buf[i, sl] = buf[i, sl] * SCALE

            pltpu.sync_copy(
                buf,
                out_hbm.at[pl.ds(base + g * CHUNK, CHUNK), pl.ds(0, D_MODEL)])

        start_gather(0, 0)
        start_gather(1, 1)

        @pl.loop(0, nchunks - 2, step=2)
        def _chunks(g0):
            for b in range(2):
                g = g0 + b
                scale_and_store(g, b)
                start_gather(g + 2, b)

        for b in range(2):
            scale_and_store(nchunks - 2 + b, b)

    return lookup


def kernel(X, table):
    rows, cols = X.shape
    B = rows * cols
    V = table.shape[0]
    xf = X.reshape(NW, B // NW).astype(jnp.int32)
    out = _make_lookup(B, V)(xf, table)
    return out[:, :D_MODEL].reshape(rows, cols, D_MODEL)
